# baseline (speedup 1.0000x reference)
import jax
import jax.numpy as jnp
from jax import lax
from jax.experimental import pallas as pl
from jax.experimental.pallas import tpu as pltpu
from jax.experimental.pallas import tpu_sc as plsc

BATCH = 16384
EMBED = 64
PAIR = 2 * EMBED
NUM_CORES = 2
NUM_SUBCORES = 16
NUM_WORKERS = NUM_CORES * NUM_SUBCORES
BPW = BATCH // NUM_WORKERS
NCHUNK = 4
CH = BPW // NCHUNK
LANES = 16
GPC = CH // LANES


def _body(uidx_hbm, midx_hbm, uemb_hbm, ubias_hbm, memb_hbm, mbias_hbm,
          out_hbm, uidx_v, midx_v, utidx_v, mtidx_v, upairs_v, mpairs_v,
          ub_v, mb_v, res_v, sem_rows, sem_misc):
    c = lax.axis_index("c")
    s = lax.axis_index("s")
    wid = s * NUM_CORES + c
    base = wid * BPW

    for j in range(NCHUNK):
        pltpu.sync_copy(uidx_hbm.at[pl.ds(base + j * CH, CH)], uidx_v.at[j])
        pltpu.sync_copy(midx_hbm.at[pl.ds(base + j * CH, CH)], midx_v.at[j])

    bias_copies = []
    for j in range(NCHUNK):
        bias_copies.append(
            pltpu.async_copy(ubias_hbm.at[uidx_v.at[j]], ub_v.at[j], sem_misc))
        bias_copies.append(
            pltpu.async_copy(mbias_hbm.at[midx_v.at[j]], mb_v.at[j], sem_misc))

    for j in range(NCHUNK):
        def tidx_body(g, carry, j=j):
            goff = pl.multiple_of(g * LANES, LANES)
            utidx_v[j, pl.ds(goff, LANES)] = lax.shift_right_logical(
                uidx_v[j, pl.ds(goff, LANES)], 1)
            mtidx_v[j, pl.ds(goff, LANES)] = lax.shift_right_logical(
                midx_v[j, pl.ds(goff, LANES)], 1)
            return carry
        lax.fori_loop(0, CH // LANES, tidx_body, 0)

    for cp in bias_copies:
        cp.wait()

    lane = lax.iota(jnp.int32, LANES)
    perms = [lane ^ sh for sh in (8, 4, 2, 1)]

    for j in range(NCHUNK):
        ucp = pltpu.async_copy(uemb_hbm.at[utidx_v.at[j]], upairs_v, sem_rows)
        mcp = pltpu.async_copy(memb_hbm.at[mtidx_v.at[j]], mpairs_v, sem_rows)
        ucp.wait()
        mcp.wait()

        def group_body(g, carry, j=j):
            goff = pl.multiple_of(g * LANES, LANES)
            uv = uidx_v[j, pl.ds(goff, LANES)]
            mv = midx_v[j, pl.ds(goff, LANES)]
            vec = jnp.zeros((LANES,), jnp.float32)
            for i in range(LANES):
                r = goff + i
                uoff = pl.multiple_of((uv[i] & 1) * EMBED, EMBED)
                moff = pl.multiple_of((mv[i] & 1) * EMBED, EMBED)
                acc = None
                for k in range(EMBED // LANES):
                    u = upairs_v[r, pl.ds(uoff + k * LANES, LANES)]
                    m = mpairs_v[r, pl.ds(moff + k * LANES, LANES)]
                    p = u * m
                    acc = p if acc is None else acc + p
                for perm in perms:
                    acc = acc + acc.at[perm].get(mode="promise_in_bounds")
                vec = jnp.where(lane == i, acc, vec)
            x = vec + ub_v[j, pl.ds(goff, LANES)] + mb_v[j, pl.ds(goff, LANES)]
            y = 1.0 / (1.0 + jnp.exp(-x))
            res_v[pl.ds(j * CH + goff, LANES)] = y
            return carry
        lax.fori_loop(0, GPC, group_body, 0)

    pltpu.sync_copy(res_v, out_hbm.at[pl.ds(base, BPW)])


@jax.jit
def _run(uidx, midx, uemb, ubias, memb, mbias):
    mesh = plsc.VectorSubcoreMesh(core_axis_name="c", subcore_axis_name="s")
    kfn = pl.kernel(
        _body,
        mesh=mesh,
        compiler_params=pltpu.CompilerParams(use_tc_tiling_on_sc=True),
        out_type=jax.ShapeDtypeStruct((BATCH,), jnp.float32),
        scratch_types=[
            pltpu.VMEM((NCHUNK, CH), jnp.int32),
            pltpu.VMEM((NCHUNK, CH), jnp.int32),
            pltpu.VMEM((NCHUNK, CH), jnp.int32),
            pltpu.VMEM((NCHUNK, CH), jnp.int32),
            pltpu.VMEM((CH, PAIR), jnp.float32),
            pltpu.VMEM((CH, PAIR), jnp.float32),
            pltpu.VMEM((NCHUNK, CH), jnp.float32),
            pltpu.VMEM((NCHUNK, CH), jnp.float32),
            pltpu.VMEM((BPW,), jnp.float32),
            pltpu.SemaphoreType.DMA,
            pltpu.SemaphoreType.DMA,
        ],
    )
    return kfn(uidx, midx, uemb, ubias, memb, mbias)


def kernel(user_input, movie_input, user_embedding, user_bias,
           movie_embedding, movie_bias):
    upairs = user_embedding.reshape(-1, PAIR)
    mpairs = movie_embedding.reshape(-1, PAIR)
    return _run(user_input.astype(jnp.int32), movie_input.astype(jnp.int32),
                upairs, user_bias.reshape(-1),
                mpairs, movie_bias.reshape(-1))


# speedup vs baseline: 1.4690x; 1.4690x over previous
import jax
import jax.numpy as jnp
from jax import lax
from jax.experimental import pallas as pl
from jax.experimental.pallas import tpu as pltpu
from jax.experimental.pallas import tpu_sc as plsc

BATCH = 16384
EMBED = 64
PAIR = 2 * EMBED
NUM_CORES = 2
NUM_SUBCORES = 16
NUM_WORKERS = NUM_CORES * NUM_SUBCORES
BPW = BATCH // NUM_WORKERS
NCHUNK = 4
CH = BPW // NCHUNK
LANES = 16
GPC = CH // LANES


def _body(uidx_hbm, midx_hbm, uemb_hbm, ubias_hbm, memb_hbm, mbias_hbm,
          out_hbm, uidx_v, midx_v, mtidx_v, urows_v, mpairs_v,
          ub_v, mb_v, res_v, sem_a, sem_b, sem_m, sem_misc):
    c = lax.axis_index("c")
    s = lax.axis_index("s")
    wid = s * NUM_CORES + c
    base = wid * BPW

    for j in range(NCHUNK):
        pltpu.sync_copy(uidx_hbm.at[pl.ds(base + j * CH, CH)], uidx_v.at[j])
        pltpu.sync_copy(midx_hbm.at[pl.ds(base + j * CH, CH)], midx_v.at[j])

    bias_copies = []
    for j in range(NCHUNK):
        bias_copies.append(
            pltpu.async_copy(ubias_hbm.at[uidx_v.at[j]], ub_v.at[j], sem_misc))
        bias_copies.append(
            pltpu.async_copy(mbias_hbm.at[midx_v.at[j]], mb_v.at[j], sem_misc))

    for j in range(NCHUNK):
        def tidx_body(g, carry, j=j):
            goff = pl.multiple_of(g * LANES, LANES)
            mtidx_v[j, pl.ds(goff, LANES)] = lax.shift_right_logical(
                midx_v[j, pl.ds(goff, LANES)], 1)
            return carry
        lax.fori_loop(0, CH // LANES, tidx_body, 0)

    for cp in bias_copies:
        cp.wait()

    lane = lax.iota(jnp.int32, LANES)
    perms = [lane ^ sh for sh in (8, 4, 2, 1)]

    for j in range(NCHUNK):
        mcp = pltpu.async_copy(memb_hbm.at[mtidx_v.at[j]], mpairs_v, sem_m)

        def fire_body(g, carry, j=j):
            goff = pl.multiple_of(g * LANES, LANES)
            uv = uidx_v[j, pl.ds(goff, LANES)]
            for i in range(LANES):
                sem = sem_a if i % 2 == 0 else sem_b
                pltpu.async_copy(uemb_hbm.at[pl.ds(uv[i], 1)],
                                 urows_v.at[pl.ds(goff + i, 1)], sem)
            return carry
        lax.fori_loop(0, GPC, fire_body, 0)

        pltpu.make_async_copy(uemb_hbm.at[pl.ds(0, CH // 2)],
                              urows_v.at[pl.ds(0, CH // 2)], sem_a).wait()
        pltpu.make_async_copy(uemb_hbm.at[pl.ds(0, CH // 2)],
                              urows_v.at[pl.ds(0, CH // 2)], sem_b).wait()
        mcp.wait()

        def group_body(g, carry, j=j):
            goff = pl.multiple_of(g * LANES, LANES)
            mv = midx_v[j, pl.ds(goff, LANES)]
            vec = jnp.zeros((LANES,), jnp.float32)
            for i in range(LANES):
                r = goff + i
                moff = pl.multiple_of((mv[i] & 1) * EMBED, EMBED)
                acc = None
                for k in range(EMBED // LANES):
                    u = urows_v[r, pl.ds(k * LANES, LANES)]
                    m = mpairs_v[r, pl.ds(moff + k * LANES, LANES)]
                    p = u * m
                    acc = p if acc is None else acc + p
                for perm in perms:
                    acc = acc + acc.at[perm].get(mode="promise_in_bounds")
                vec = jnp.where(lane == i, acc, vec)
            x = vec + ub_v[j, pl.ds(goff, LANES)] + mb_v[j, pl.ds(goff, LANES)]
            y = 1.0 / (1.0 + jnp.exp(-x))
            res_v[pl.ds(j * CH + goff, LANES)] = y
            return carry
        lax.fori_loop(0, GPC, group_body, 0)

    pltpu.sync_copy(res_v, out_hbm.at[pl.ds(base, BPW)])


@jax.jit
def _run(uidx, midx, uemb, ubias, memb, mbias):
    mesh = plsc.VectorSubcoreMesh(core_axis_name="c", subcore_axis_name="s")
    kfn = pl.kernel(
        _body,
        mesh=mesh,
        compiler_params=pltpu.CompilerParams(use_tc_tiling_on_sc=True),
        out_type=jax.ShapeDtypeStruct((BATCH,), jnp.float32),
        scratch_types=[
            pltpu.VMEM((NCHUNK, CH), jnp.int32),
            pltpu.VMEM((NCHUNK, CH), jnp.int32),
            pltpu.VMEM((NCHUNK, CH), jnp.int32),
            pltpu.VMEM((CH, EMBED), jnp.float32),
            pltpu.VMEM((CH, PAIR), jnp.float32),
            pltpu.VMEM((NCHUNK, CH), jnp.float32),
            pltpu.VMEM((NCHUNK, CH), jnp.float32),
            pltpu.VMEM((BPW,), jnp.float32),
            pltpu.SemaphoreType.DMA,
            pltpu.SemaphoreType.DMA,
            pltpu.SemaphoreType.DMA,
            pltpu.SemaphoreType.DMA,
        ],
    )
    return kfn(uidx, midx, uemb, ubias, memb, mbias)


def kernel(user_input, movie_input, user_embedding, user_bias,
           movie_embedding, movie_bias):
    mpairs = movie_embedding.reshape(-1, PAIR)
    return _run(user_input.astype(jnp.int32), movie_input.astype(jnp.int32),
                user_embedding, user_bias.reshape(-1),
                mpairs, movie_bias.reshape(-1))
